# baseline (speedup 1.0000x reference)
import functools

import jax
import jax.numpy as jnp
from jax import lax
from jax.experimental import pallas as pl
from jax.experimental.pallas import tpu as pltpu
from jax.experimental.pallas import tpu_sc as plsc

B = 8
H = 512
W = 512
HW = H * W
NC = 2
NS = 16
L = 16

B_PER_SC = B // NC
ROWS_PER_TEC = H // NS
RB = 8
N_BLK = ROWS_PER_TEC // RB
SUB = 2048
ZCH = 4096
BIAS = 4096
PAD = 640
ENDPAD = 1152
SPMEM_WORDS = PAD + B_PER_SC * HW + ENDPAD


def _make_kernel():
    mesh = plsc.VectorSubcoreMesh(
        core_axis_name="c", subcore_axis_name="s", num_cores=NC, num_subcores=NS
    )

    @functools.partial(
        pl.kernel,
        out_type=jax.ShapeDtypeStruct((B * HW,), jnp.float32),
        mesh=mesh,
        scratch_types=[
            [pltpu.VMEM((RB, W), jnp.float32)] * 2,
            [pltpu.VMEM((RB, W), jnp.float32)] * 2,
            [pltpu.VMEM((4 * SUB,), jnp.int32)] * 2,
            [pltpu.VMEM((4 * SUB,), jnp.float32)] * 2,
            pltpu.VMEM((W,), jnp.float32),
            pltpu.VMEM((ZCH,), jnp.float32),
            pltpu.VMEM_SHARED((SPMEM_WORDS,), jnp.float32),
            [pltpu.SemaphoreType.DMA] * 2,
            [pltpu.SemaphoreType.DMA] * 2,
        ],
        compiler_params=pltpu.CompilerParams(use_tc_tiling_on_sc=True),
    )
    def splat(flow_hbm, out_hbm, ubuf, vbuf, idxb, valb, xfb, zbuf, spmem,
              isem, ssem):
        c = lax.axis_index("c")
        s = lax.axis_index("s")

        lane = lax.iota(jnp.int32, L)
        fbias = jnp.float32(BIAS)

        def _tfill(i, carry):
            jj = i * L
            xfb[pl.ds(jj, L)] = (lane + jj).astype(jnp.float32) + fbias
            return carry

        lax.fori_loop(0, W // L, _tfill, 0)

        def _zfill(i, carry):
            zbuf[pl.ds(i * L, L)] = jnp.zeros((L,), jnp.float32)
            return carry

        lax.fori_loop(0, ZCH // L, _zfill, 0)
        words_per_tec = (B_PER_SC * HW) // NS
        for t in range(words_per_tec // ZCH):
            pltpu.sync_copy(
                zbuf, spmem.at[pl.ds(PAD + s * words_per_tec + t * ZCH, ZCH)])
        plsc.subcore_barrier()

        NBT = B_PER_SC * N_BLK

        def _start_in(t, buf):
            l, k = divmod(t, N_BLK)
            b = 2 * l + c
            r0 = s * ROWS_PER_TEC + k * RB
            du = pltpu.async_copy(
                flow_hbm.at[b, 0, pl.ds(r0, RB), :], ubuf[buf], isem[buf])
            dv = pltpu.async_copy(
                flow_hbm.at[b, 1, pl.ds(r0, RB), :], vbuf[buf], isem[buf])
            return du, dv

        in_d = [None, None]
        sc_d = [None, None]
        in_d[0] = _start_in(0, 0)
        for t in range(NBT):
            cur = t % 2
            nxt = (t + 1) % 2
            if t + 1 < NBT:
                in_d[nxt] = _start_in(t + 1, nxt)
            du, dv = in_d[cur]
            du.wait()
            dv.wait()
            l, k = divmod(t, N_BLK)
            r0 = s * ROWS_PER_TEC + k * RB
            koff = PAD + l * HW - BIAS * W - BIAS
            r0f = jnp.float32(1.0) * r0 + fbias

            for half in range(2):
                hbuf = (2 * t + half) % 2
                if sc_d[hbuf] is not None:
                    sc_d[hbuf].wait()

                def _compute(i, carry, cur=cur, hbuf=hbuf, half=half,
                             r0f=r0f, koff=koff):
                    rr = jnp.right_shift(i, 5) + 4 * half
                    cc = pl.multiple_of(
                        jnp.left_shift(jnp.bitwise_and(i, 31), 4), L)
                    q = pl.multiple_of(i * L, L)
                    u = ubuf[cur][rr, pl.ds(cc, L)]
                    v = vbuf[cur][rr, pl.ds(cc, L)]
                    gx = u + xfb[pl.ds(cc, L)]
                    gy = v + (r0f + rr.astype(jnp.float32))
                    itx = gx.astype(jnp.int32)
                    ity = gy.astype(jnp.int32)
                    ax = gx - itx.astype(jnp.float32)
                    ay = gy - ity.astype(jnp.float32)
                    bx = 1.0 - ax
                    by = 1.0 - ay
                    ex = (itx - (BIAS - 1)).astype(jnp.uint32)
                    ey = (ity - (BIAS - 1)).astype(jnp.uint32)
                    vx1 = ex < jnp.uint32(W)
                    vx0 = (ex - 1) < jnp.uint32(W)
                    vy1 = ey < jnp.uint32(H)
                    vy0 = (ey - 1) < jnp.uint32(H)
                    zero = jnp.zeros((L,), jnp.float32)
                    axm = jnp.where(vx1, ax, zero)
                    bxm = jnp.where(vx0, bx, zero)
                    aym = jnp.where(vy1, ay, zero)
                    bym = jnp.where(vy0, by, zero)
                    ctx = jnp.clip(itx, BIAS - 1, BIAS + W - 1)
                    cty = jnp.clip(ity, BIAS - 1, BIAS + H - 1)
                    i00 = jnp.left_shift(cty, 9) + (ctx + koff)
                    idxb[hbuf][pl.ds(0 * SUB + q, L)] = i00
                    valb[hbuf][pl.ds(0 * SUB + q, L)] = bxm * bym
                    idxb[hbuf][pl.ds(1 * SUB + q, L)] = i00 + 1
                    valb[hbuf][pl.ds(1 * SUB + q, L)] = axm * bym
                    idxb[hbuf][pl.ds(2 * SUB + q, L)] = i00 + W
                    valb[hbuf][pl.ds(2 * SUB + q, L)] = bxm * aym
                    idxb[hbuf][pl.ds(3 * SUB + q, L)] = i00 + (W + 1)
                    valb[hbuf][pl.ds(3 * SUB + q, L)] = axm * aym
                    return carry

                lax.fori_loop(0, SUB // L, _compute, 0)
                sc_d[hbuf] = pltpu.async_copy(
                    valb[hbuf], spmem.at[idxb[hbuf]], ssem[hbuf], add=True)
        for d in sc_d:
            if d is not None:
                d.wait()

        plsc.subcore_barrier()

        for l in range(B_PER_SC):
            b = 2 * l + c
            src = PAD + l * HW + s * (ROWS_PER_TEC * W)
            pltpu.sync_copy(
                spmem.at[pl.ds(src, ROWS_PER_TEC * W)],
                out_hbm.at[pl.ds(b * HW + s * (ROWS_PER_TEC * W), ROWS_PER_TEC * W)],
            )

    return splat


_splat = _make_kernel()


def kernel(img, flow):
    del img
    return _splat(flow).reshape(B, 1, H, W)


# speedup vs baseline: 1.0139x; 1.0139x over previous
import functools

import jax
import jax.numpy as jnp
from jax import lax
from jax.experimental import pallas as pl
from jax.experimental.pallas import tpu as pltpu
from jax.experimental.pallas import tpu_sc as plsc

B = 8
H = 512
W = 512
HW = H * W
NC = 2
NS = 16
L = 16

B_PER_SC = B // NC
ROWS_PER_TEC = H // NS
RB = 8
N_BLK = ROWS_PER_TEC // RB
SUB = 2048
ZCH = 4096
BIAS = 4096
PAD = 640
ENDPAD = 1152
SPMEM_WORDS = PAD + B_PER_SC * HW + ENDPAD


def _make_kernel():
    mesh = plsc.VectorSubcoreMesh(
        core_axis_name="c", subcore_axis_name="s", num_cores=NC, num_subcores=NS
    )

    @functools.partial(
        pl.kernel,
        out_type=jax.ShapeDtypeStruct((B * HW,), jnp.float32),
        mesh=mesh,
        scratch_types=[
            [pltpu.VMEM((RB, W), jnp.float32)] * 2,
            [pltpu.VMEM((RB, W), jnp.float32)] * 2,
            [pltpu.VMEM((4 * SUB,), jnp.int32)] * 2,
            [pltpu.VMEM((4 * SUB,), jnp.float32)] * 2,
            pltpu.VMEM((W,), jnp.float32),
            pltpu.VMEM((ZCH,), jnp.float32),
            pltpu.VMEM_SHARED((SPMEM_WORDS,), jnp.float32),
            [pltpu.SemaphoreType.DMA] * 2,
            [pltpu.SemaphoreType.DMA] * 2,
        ],
        compiler_params=pltpu.CompilerParams(use_tc_tiling_on_sc=True),
    )
    def splat(flow_hbm, out_hbm, ubuf, vbuf, idxb, valb, xfb, zbuf, spmem,
              isem, ssem):
        c = lax.axis_index("c")
        s = lax.axis_index("s")

        lane = lax.iota(jnp.int32, L)
        fbias = jnp.float32(BIAS)

        def _tfill(i, carry):
            jj = i * L
            xfb[pl.ds(jj, L)] = (lane + jj).astype(jnp.float32) + fbias
            return carry

        lax.fori_loop(0, W // L, _tfill, 0)

        def _zfill(i, carry):
            zbuf[pl.ds(i * L, L)] = jnp.zeros((L,), jnp.float32)
            return carry

        lax.fori_loop(0, ZCH // L, _zfill, 0)
        words_per_tec = (B_PER_SC * HW) // NS
        for t in range(words_per_tec // ZCH):
            pltpu.sync_copy(
                zbuf, spmem.at[pl.ds(PAD + s * words_per_tec + t * ZCH, ZCH)])
        plsc.subcore_barrier()

        NBT = B_PER_SC * N_BLK

        def _start_in(t, buf):
            l, k = divmod(t, N_BLK)
            b = 2 * l + c
            r0 = s * ROWS_PER_TEC + k * RB
            du = pltpu.async_copy(
                flow_hbm.at[b, 0, pl.ds(r0, RB), :], ubuf[buf], isem[buf])
            dv = pltpu.async_copy(
                flow_hbm.at[b, 1, pl.ds(r0, RB), :], vbuf[buf], isem[buf])
            return du, dv

        in_d = [None, None]
        sc_d = [None, None]
        in_d[0] = _start_in(0, 0)
        for t in range(NBT):
            cur = t % 2
            nxt = (t + 1) % 2
            if t + 1 < NBT:
                in_d[nxt] = _start_in(t + 1, nxt)
            du, dv = in_d[cur]
            du.wait()
            dv.wait()
            l, k = divmod(t, N_BLK)
            r0 = s * ROWS_PER_TEC + k * RB
            koff = PAD + l * HW - BIAS * W - BIAS
            r0f = jnp.float32(1.0) * r0 + fbias

            for half in range(2):
                hbuf = (2 * t + half) % 2
                if sc_d[hbuf] is not None:
                    sc_d[hbuf].wait()

                def _compute(i, carry, cur=cur, hbuf=hbuf, half=half,
                             r0f=r0f, koff=koff):
                    rr = jnp.right_shift(i, 5) + 4 * half
                    cc = pl.multiple_of(
                        jnp.left_shift(jnp.bitwise_and(i, 31), 4), L)
                    q = pl.multiple_of(i * L, L)
                    u = ubuf[cur][rr, pl.ds(cc, L)]
                    v = vbuf[cur][rr, pl.ds(cc, L)]
                    gx = u + xfb[pl.ds(cc, L)]
                    gy = v + (r0f + rr.astype(jnp.float32))
                    itx = gx.astype(jnp.int32)
                    ity = gy.astype(jnp.int32)
                    ax = gx - itx.astype(jnp.float32)
                    ay = gy - ity.astype(jnp.float32)
                    bx = 1.0 - ax
                    by = 1.0 - ay
                    ex = (itx - (BIAS - 1)).astype(jnp.uint32)
                    ey = (ity - (BIAS - 1)).astype(jnp.uint32)
                    vx1 = ex < jnp.uint32(W)
                    vx0 = (ex - 1) < jnp.uint32(W)
                    vy1 = ey < jnp.uint32(H)
                    vy0 = (ey - 1) < jnp.uint32(H)
                    zero = jnp.zeros((L,), jnp.float32)
                    axm = jnp.where(vx1, ax, zero)
                    bxm = jnp.where(vx0, bx, zero)
                    aym = jnp.where(vy1, ay, zero)
                    bym = jnp.where(vy0, by, zero)
                    ctx = jnp.clip(itx, BIAS - 1, BIAS + W - 1)
                    cty = jnp.clip(ity, BIAS - 1, BIAS + H - 1)
                    i00 = jnp.left_shift(cty, 9) + (ctx + koff)
                    idxb[hbuf][pl.ds(0 * SUB + q, L)] = i00
                    valb[hbuf][pl.ds(0 * SUB + q, L)] = bxm * bym
                    idxb[hbuf][pl.ds(1 * SUB + q, L)] = i00 + 1
                    valb[hbuf][pl.ds(1 * SUB + q, L)] = axm * bym
                    idxb[hbuf][pl.ds(2 * SUB + q, L)] = i00 + W
                    valb[hbuf][pl.ds(2 * SUB + q, L)] = bxm * aym
                    idxb[hbuf][pl.ds(3 * SUB + q, L)] = i00 + (W + 1)
                    valb[hbuf][pl.ds(3 * SUB + q, L)] = axm * aym
                    return carry

                def _pbody(i, _compute=_compute):
                    _compute(i, 0)

                plsc.parallel_loop(0, SUB // L, 1, unroll=2)(_pbody)
                sc_d[hbuf] = pltpu.async_copy(
                    valb[hbuf], spmem.at[idxb[hbuf]], ssem[hbuf], add=True)
        for d in sc_d:
            if d is not None:
                d.wait()

        plsc.subcore_barrier()

        for l in range(B_PER_SC):
            b = 2 * l + c
            src = PAD + l * HW + s * (ROWS_PER_TEC * W)
            pltpu.sync_copy(
                spmem.at[pl.ds(src, ROWS_PER_TEC * W)],
                out_hbm.at[pl.ds(b * HW + s * (ROWS_PER_TEC * W), ROWS_PER_TEC * W)],
            )

    return splat


_splat = _make_kernel()


def kernel(img, flow):
    del img
    return _splat(flow).reshape(B, 1, H, W)
